# baseline (speedup 1.0000x reference)
import functools

import jax
import jax.numpy as jnp
from jax import lax
from jax.experimental import pallas as pl
from jax.experimental.pallas import tpu as pltpu
from jax.experimental.pallas import tpu_sc as plsc

_LEVELS = (100, 50, 25, 13)
_STARTS = (0, 10000, 12500, 13125)
_N, _Q, _H, _C = 2, 13294, 8, 32
_S = 13294
_R = _N * _Q * _H
_NB = _Q * _H
_NW = 32
_SUB = 8
_CHROWS = 64
_NFULL = _R // _CHROWS
_TAILROWS = _R - _NFULL * _CHROWS
_G4 = _N * _S * _H // 4

_MAGIC = 12582912.0


def _consts():
  i32 = jnp.int32
  w0, w1, w2, w3 = [float(w) for w in _LEVELS]
  lane = lax.iota(i32, 16)
  lo = lane < 8
  sca = jnp.where(lo, w0, w1).astype(jnp.float32)
  scb = jnp.where(lo, w2, w3).astype(jnp.float32)
  odd = (lane & 1) == 1
  wfa = jnp.where(odd, sca, 1.0)
  wfb = jnp.where(odd, scb, 1.0)
  starth2 = jnp.where(
      lane < 4, _STARTS[0] * 2,
      jnp.where(lane < 8, _STARTS[1] * 2,
                jnp.where(lane < 12, _STARTS[2] * 2, _STARTS[3] * 2)),
  ).astype(i32)
  shift = jnp.minimum(lane + 1, 15)
  ev = (lane * 2) & 15
  zero = lane * 0
  bcast = [zero + i for i in range(16)]
  return sca, scb, wfa, wfb, starth2, shift, ev, lo, bcast


def _body(value_hbm, loc_hbm, attn_hbm, out_hbm,
          loc_v, attn_v, w_v, idx_v, gat_v, out_v, sem):
  sca, scb, wfa, wfb, starth2, shift, ev, lane_lo, bcast = _consts()
  wid = lax.axis_index("s") * 2 + lax.axis_index("c")
  nchunks_w = (_NFULL - 1 - wid) // _NW + 1

  def tka(v, idx):
    return jnp.take_along_axis(v, idx, axis=0)

  def do_chunk(c, row0, nsub):
    pltpu.sync_copy(loc_hbm.at[pl.ds(c * (_CHROWS // 4), nsub * 2)],
                    loc_v.at[pl.ds(0, nsub * 2)])
    pltpu.sync_copy(attn_hbm.at[pl.ds(c * (_CHROWS // 8), nsub)],
                    attn_v.at[pl.ds(0, nsub)])

    def sub1(r2, c1):
      for r in range(_SUB):
        lrow = r2 * 2 + r // 4
        lcol = (r % 4) * 32
        a = loc_v[lrow, pl.ds(lcol, 16)]
        b = loc_v[lrow, pl.ds(lcol + 16, 16)]
        ta = ((a * sca - 0.5) + _MAGIC) - _MAGIC
        tb = ((b * scb - 0.5) + _MAGIC) - _MAGIC
        va = jnp.where((ta >= 0.0) & (ta < sca), 1.0, 0.0).astype(jnp.float32)
        vb = jnp.where((tb >= 0.0) & (tb < scb), 1.0, 0.0).astype(jnp.float32)
        ca = jnp.clip(ta, 0.0, sca - 1.0) * wfa
        cb = jnp.clip(tb, 0.0, scb - 1.0) * wfb
        sa = ca + tka(ca, shift)
        sb = cb + tka(cb, shift)
        pa = va * tka(va, shift)
        pb = vb * tka(vb, shift)
        s16 = jnp.where(lane_lo, tka(sa, ev), tka(sb, ev))
        p16 = jnp.where(lane_lo, tka(pa, ev), tka(pb, ev))
        row = row0 + r2 * _SUB + r
        n1 = (row >= _NB).astype(jnp.int32)
        gidx = s16.astype(jnp.int32) * 2 + starth2 + (
            n1 * (_S * _H // 4) + (r >> 2))
        idx_v[r2, pl.ds(r * 16, 16)] = gidx
        w_v[pl.ds(r2 * 128 + r * 16, 16)] = attn_v[r2, pl.ds(r * 16, 16)] * p16
      return c1

    lax.fori_loop(0, nsub, sub1, 0)

    for wave in range(nsub // 4):
      copies = [
          pltpu.async_copy(value_hbm.at[idx_v.at[wave * 4 + j]], gat_v.at[j],
                           sem)
          for j in range(4)
      ]
      for cp in copies:
        cp.wait()

      def sub2(r2, c2):
        sb = wave * 4 + r2
        for r in range(_SUB):
          off = (r & 3) * 32
          orow = sb * 2 + r // 4
          ocol = (r % 4) * 32
          w16 = w_v[pl.ds(sb * 128 + r * 16, 16)]
          wi = tka(w16, bcast[0])
          acc0 = wi * gat_v[r2, 0 + r * 16, off:off + 16]
          acc1 = wi * gat_v[r2, 0 + r * 16, off + 16:off + 32]
          for i in range(1, 16):
            wi = tka(w16, bcast[i])
            acc0 = acc0 + wi * gat_v[r2, r * 16 + i, off:off + 16]
            acc1 = acc1 + wi * gat_v[r2, r * 16 + i, off + 16:off + 32]
          out_v[orow, pl.ds(ocol, 16)] = acc0
          out_v[orow, pl.ds(ocol + 16, 16)] = acc1
        return c2

      lax.fori_loop(0, 4, sub2, 0)

    pltpu.sync_copy(out_v.at[pl.ds(0, nsub * 2)],
                    out_hbm.at[pl.ds(c * (_CHROWS // 4), nsub * 2)])

  def chunk(k, carry):
    c = wid + k * _NW
    do_chunk(c, c * _CHROWS, _CHROWS // _SUB)
    return carry

  lax.fori_loop(0, nchunks_w, chunk, 0)

  @pl.when(wid == _NFULL % _NW)
  def _tail():
    do_chunk(_NFULL, _NFULL * _CHROWS, _TAILROWS // _SUB)


@jax.jit
def _run(value4, loc2, attn2):
  kfn = pl.kernel(
      _body,
      out_type=jax.ShapeDtypeStruct((_R // 4, 128), jnp.float32),
      mesh=plsc.VectorSubcoreMesh(core_axis_name="c", subcore_axis_name="s"),
      scratch_types=[
          pltpu.VMEM((_CHROWS // 4, 128), jnp.float32),
          pltpu.VMEM((_CHROWS // 8, 128), jnp.float32),
          pltpu.VMEM((_CHROWS * 16,), jnp.float32),
          pltpu.VMEM((_CHROWS // 8, 128), jnp.int32),
          pltpu.VMEM((4, _SUB * 16, 128), jnp.float32),
          pltpu.VMEM((_CHROWS // 4, 128), jnp.float32),
          pltpu.SemaphoreType.DMA,
      ],
      compiler_params=pltpu.CompilerParams(use_tc_tiling_on_sc=True),
  )
  return kfn(value4, loc2, attn2)


def kernel(value, spatial_shapes, level_start_index, sampling_loc, attn_weight):
  N, S, H, C = value.shape
  one = (spatial_shapes[0, 0] // spatial_shapes[0, 0]).astype(value.dtype)
  value4 = value.reshape(_G4, 128) * one
  loc2 = sampling_loc.reshape(_R // 4, 128)
  attn2 = attn_weight.reshape(_R // 8, 128)
  out = _run(value4, loc2, attn2)
  return out.reshape(_N, _Q, _H, _C)


# speedup vs baseline: 1.0047x; 1.0047x over previous
import functools

import jax
import jax.numpy as jnp
from jax import lax
from jax.experimental import pallas as pl
from jax.experimental.pallas import tpu as pltpu
from jax.experimental.pallas import tpu_sc as plsc

_LEVELS = (100, 50, 25, 13)
_STARTS = (0, 10000, 12500, 13125)
_N, _Q, _H, _C = 2, 13294, 8, 32
_S = 13294
_R = _N * _Q * _H
_NB = _Q * _H
_NW = 32
_SUB = 8
_CHROWS = 64
_NFULL = _R // _CHROWS
_TAILROWS = _R - _NFULL * _CHROWS
_G4 = _N * _S * _H // 4

_MAGIC = 12582912.0


def _consts():
  i32 = jnp.int32
  w0, w1, w2, w3 = [float(w) for w in _LEVELS]
  lane = lax.iota(i32, 16)
  lo = lane < 8
  sca = jnp.where(lo, w0, w1).astype(jnp.float32)
  scb = jnp.where(lo, w2, w3).astype(jnp.float32)
  odd = (lane & 1) == 1
  wfa = jnp.where(odd, sca, 1.0)
  wfb = jnp.where(odd, scb, 1.0)
  starth2 = jnp.where(
      lane < 4, _STARTS[0] * 2,
      jnp.where(lane < 8, _STARTS[1] * 2,
                jnp.where(lane < 12, _STARTS[2] * 2, _STARTS[3] * 2)),
  ).astype(i32)
  shift = jnp.minimum(lane + 1, 15)
  ev = (lane * 2) & 15
  zero = lane * 0
  bcast = [zero + i for i in range(16)]
  return sca, scb, wfa, wfb, starth2, shift, ev, lo, bcast


def _body(value_hbm, loc_hbm, attn_hbm, out_hbm,
          loc_v, attn_v, w_v, idx_v, gat_v, out_v, sem):
  sca, scb, wfa, wfb, starth2, shift, ev, lane_lo, bcast = _consts()
  wid = lax.axis_index("s") * 2 + lax.axis_index("c")
  nchunks_w = (_NFULL - 1 - wid) // _NW + 1

  def tka(v, idx):
    return jnp.take_along_axis(v, idx, axis=0)

  def do_chunk(c, row0, nsub):
    pltpu.sync_copy(loc_hbm.at[pl.ds(c * (_CHROWS // 4), nsub * 2)],
                    loc_v.at[pl.ds(0, nsub * 2)])
    pltpu.sync_copy(attn_hbm.at[pl.ds(c * (_CHROWS // 8), nsub)],
                    attn_v.at[pl.ds(0, nsub)])

    def sub1(r2, c1):
      for r in range(_SUB):
        lrow = r2 * 2 + r // 4
        lcol = (r % 4) * 32
        a = loc_v[lrow, pl.ds(lcol, 16)]
        b = loc_v[lrow, pl.ds(lcol + 16, 16)]
        ta = ((a * sca - 0.5) + _MAGIC) - _MAGIC
        tb = ((b * scb - 0.5) + _MAGIC) - _MAGIC
        va = jnp.where((ta >= 0.0) & (ta < sca), 1.0, 0.0).astype(jnp.float32)
        vb = jnp.where((tb >= 0.0) & (tb < scb), 1.0, 0.0).astype(jnp.float32)
        ca = jnp.clip(ta, 0.0, sca - 1.0) * wfa
        cb = jnp.clip(tb, 0.0, scb - 1.0) * wfb
        sa = ca + tka(ca, shift)
        sb = cb + tka(cb, shift)
        pa = va * tka(va, shift)
        pb = vb * tka(vb, shift)
        s16 = jnp.where(lane_lo, tka(sa, ev), tka(sb, ev))
        p16 = jnp.where(lane_lo, tka(pa, ev), tka(pb, ev))
        row = row0 + r2 * _SUB + r
        n1 = (row >= _NB).astype(jnp.int32)
        gidx = s16.astype(jnp.int32) * 2 + starth2 + (
            n1 * (_S * _H // 4) + (r >> 2))
        idx_v[r2, pl.ds(r * 16, 16)] = gidx
        w_v[pl.ds(r2 * 128 + r * 16, 16)] = attn_v[r2, pl.ds(r * 16, 16)] * p16
      return c1

    lax.fori_loop(0, nsub, sub1, 0)

    for wave in range(nsub // 4):
      copies = [
          pltpu.async_copy(value_hbm.at[idx_v.at[wave * 4 + j]], gat_v.at[j],
                           sem)
          for j in range(4)
      ]
      for cp in copies:
        cp.wait()

      def sub2(r2, c2):
        sb = wave * 4 + r2
        for r in range(_SUB):
          off = (r & 3) * 32
          orow = sb * 2 + r // 4
          ocol = (r % 4) * 32
          w16 = w_v[pl.ds(sb * 128 + r * 16, 16)]
          wi = tka(w16, bcast[0])
          acc0 = wi * gat_v[r2, 0 + r * 16, off:off + 16]
          acc1 = wi * gat_v[r2, 0 + r * 16, off + 16:off + 32]
          for i in range(1, 16):
            wi = tka(w16, bcast[i])
            acc0 = acc0 + wi * gat_v[r2, r * 16 + i, off:off + 16]
            acc1 = acc1 + wi * gat_v[r2, r * 16 + i, off + 16:off + 32]
          out_v[orow, pl.ds(ocol, 16)] = acc0
          out_v[orow, pl.ds(ocol + 16, 16)] = acc1
        return c2

      lax.fori_loop(0, 4, sub2, 0)

    pltpu.sync_copy(out_v.at[pl.ds(0, nsub * 2)],
                    out_hbm.at[pl.ds(c * (_CHROWS // 4), nsub * 2)])

  def chunk(k, carry):
    c = wid + k * _NW
    do_chunk(c, c * _CHROWS, _CHROWS // _SUB)
    return carry

  lax.fori_loop(0, nchunks_w, chunk, 0)

  @pl.when(wid == _NFULL % _NW)
  def _tail():
    do_chunk(_NFULL, _NFULL * _CHROWS, _TAILROWS // _SUB)


@jax.jit
def _run(value4, loc2, attn2):
  kfn = pl.kernel(
      _body,
      out_type=jax.ShapeDtypeStruct((_R // 4, 128), jnp.float32),
      mesh=plsc.VectorSubcoreMesh(core_axis_name="c", subcore_axis_name="s"),
      scratch_types=[
          pltpu.VMEM((_CHROWS // 4, 128), jnp.float32),
          pltpu.VMEM((_CHROWS // 8, 128), jnp.float32),
          pltpu.VMEM((_CHROWS * 16,), jnp.float32),
          pltpu.VMEM((_CHROWS // 8, 128), jnp.int32),
          pltpu.VMEM((4, _SUB * 16, 128), jnp.float32),
          pltpu.VMEM((_CHROWS // 4, 128), jnp.float32),
          pltpu.SemaphoreType.DMA,
      ],
      compiler_params=pltpu.CompilerParams(use_tc_tiling_on_sc=True),
  )
  return kfn(value4, loc2, attn2)


def kernel(value, spatial_shapes, level_start_index, sampling_loc, attn_weight):
  N, S, H, C = value.shape
  value4 = value.reshape(_G4, 128) @ jnp.eye(128, dtype=value.dtype)
  loc2 = sampling_loc.reshape(_R // 4, 128)
  attn2 = attn_weight.reshape(_R // 8, 128)
  out = _run(value4, loc2, attn2)
  return out.reshape(_N, _Q, _H, _C)
